# Initial kernel scaffold; baseline (speedup 1.0000x reference)
import jax
import jax.numpy as jnp
from jax.experimental import pallas as pl


def kernel(h, edge_index, s_node, W, A):
    raise NotImplementedError("write your pallas kernel here")


# speedup vs baseline: 34.0951x; 34.0951x over previous
import functools

import jax
import jax.numpy as jnp
from jax import lax
from jax.experimental import pallas as pl
from jax.experimental.pallas import tpu as pltpu
from jax.experimental.pallas import tpu_sc as plsc

N_NODES = 10000
N_EDGES = 320000
IN_DIM = 128
OUT_DIM = 32
NUM_HEADS = 4
N_SNODE = 5000
D = NUM_HEADS * OUT_DIM

NC = 2
NS = 16
NW = NC * NS
EW = N_EDGES // NW
C = 80
NCHUNK = EW // C
NPAD = 10240
RPT = NPAD // NS
ZR = 64
DEN = NUM_HEADS * NPAD
DENT = DEN // NS
AE = 8 * N_NODES
AET = AE // NS
AVST = 1000

SN_PAD = 5120
SNW = SN_PAD // NW

_mesh = functools.partial(
    plsc.VectorSubcoreMesh, core_axis_name="c", subcore_axis_name="s",
    num_cores=NC, num_subcores=NS)

_SC_PARAMS = pltpu.CompilerParams(needs_layout_passes=False)

_Z16 = functools.partial(jnp.zeros, (16,), jnp.float32)


def _tc_prep_body(h_ref, wt_ref, ab_ref, z_ref, a_ref):
  z = jnp.dot(h_ref[...], wt_ref[...], preferred_element_type=jnp.float32)
  z_ref[...] = z
  a_ref[...] = jnp.dot(z, ab_ref[...], preferred_element_type=jnp.float32).T


def _tc_prep(h, wt, ab):
  return pl.pallas_call(
      _tc_prep_body,
      out_shape=(
          jax.ShapeDtypeStruct((N_NODES, D), jnp.float32),
          jax.ShapeDtypeStruct((8, N_NODES), jnp.float32),
      ),
  )(h, wt, ab)


def _full16(v):
  return jnp.full((16,), v, jnp.int32)


def _sc_edge_body(src_hbm, dst_hbm, a_hbm, z_hbm, shp_hbm, denp_hbm,
                  srcv, dstv, idxb, rows_v, p_v, av_g, zb1, zb2, avst,
                  a_sp, sh_sp, den_sp, semz, sema):
  c = lax.axis_index("c")
  s = lax.axis_index("s")
  wid = s * NC + c

  def _zero_body(i, _):
    for j in range(8):
      zb1[i, pl.ds(16 * j, 16)] = _Z16()
    return 0
  lax.fori_loop(0, ZR, _zero_body, 0)

  def _zero_flat(i, _):
    zb2[pl.ds(16 * i, 16)] = _Z16()
    return 0
  lax.fori_loop(0, DENT // 16, _zero_flat, 0)

  for k in range(RPT // ZR):
    pltpu.sync_copy(zb1, sh_sp.at[pl.ds(RPT * s + ZR * k, ZR)])
  pltpu.sync_copy(zb2, den_sp.at[pl.ds(DENT * s, DENT)])

  for k in range(AET // AVST):
    a0 = AET * s + AVST * k
    pltpu.sync_copy(a_hbm.at[pl.ds(a0, AVST)], avst)
    pltpu.sync_copy(avst, a_sp.at[pl.ds(a0, AVST)])
  plsc.subcore_barrier()

  base = wid * EW

  def _chunk_body(i, _):
    off = base + i * C
    pltpu.sync_copy(src_hbm.at[pl.ds(off, C)], srcv)
    pltpu.sync_copy(dst_hbm.at[pl.ds(off, C)], dstv)
    cz = pltpu.async_copy(z_hbm.at[srcv], rows_v, semz)

    for b in range(8):
      basev = srcv if b < 4 else dstv
      def _mkidx(g, _, b=b, basev=basev):
        idxb[pl.ds(16 * g, 16)] = basev[pl.ds(16 * g, 16)] + b * N_NODES
        return 0
      lax.fori_loop(0, C // 16, _mkidx, 0)
      pltpu.async_copy(a_sp.at[idxb], av_g.at[pl.ds(C * b, C)], sema).wait()

    for g in range(C // 16):
      for hh in range(NUM_HEADS):
        u = (av_g[pl.ds(C * hh + 16 * g, 16)] +
             av_g[pl.ds(C * (4 + hh) + 16 * g, 16)])
        p_v[pl.ds(C * hh + 16 * g, 16)] = jnp.exp(
            jnp.where(u >= 0, u, 0.01 * u))

    cz.wait()

    def _edge_body(e, _):
      for hh in range(NUM_HEADS):
        pb = plsc.load_gather(p_v, [_full16(C * hh) + e])
        for k in range(2):
          sl = pl.ds(32 * hh + 16 * k, 16)
          rows_v[e, sl] = rows_v[e, sl] * pb
      return 0
    lax.fori_loop(0, C, _edge_body, 0)

    pltpu.sync_copy(rows_v, sh_sp.at[dstv], add=True)
    for hh in range(NUM_HEADS):
      def _mkidx2(g, _, hh=hh):
        idxb[pl.ds(16 * g, 16)] = dstv[pl.ds(16 * g, 16)] + hh * NPAD
        return 0
      lax.fori_loop(0, C // 16, _mkidx2, 0)
      pltpu.sync_copy(p_v.at[pl.ds(C * hh, C)], den_sp.at[idxb], add=True)
    return 0
  lax.fori_loop(0, NCHUNK, _chunk_body, 0)

  plsc.subcore_barrier()

  for k in range(RPT // ZR):
    r0 = RPT * s + ZR * k
    pltpu.sync_copy(sh_sp.at[pl.ds(r0, ZR)], zb1)
    pltpu.sync_copy(zb1, shp_hbm.at[pl.ds(c * NPAD + r0, ZR)])
  pltpu.sync_copy(den_sp.at[pl.ds(DENT * s, DENT)], zb2)
  pltpu.sync_copy(zb2, denp_hbm.at[pl.ds(c * DEN + DENT * s, DENT)])


def _sc_edge(src, dst, a_flat, z_all):
  kern = pl.kernel(
      _sc_edge_body,
      out_type=(
          jax.ShapeDtypeStruct((NC * NPAD, D), jnp.float32),
          jax.ShapeDtypeStruct((NC * DEN,), jnp.float32),
      ),
      mesh=_mesh(),
      compiler_params=_SC_PARAMS,
      scratch_types=[
          pltpu.VMEM((C,), jnp.int32),
          pltpu.VMEM((C,), jnp.int32),
          pltpu.VMEM((C,), jnp.int32),
          pltpu.VMEM((C, D), jnp.float32),
          pltpu.VMEM((NUM_HEADS * C,), jnp.float32),
          pltpu.VMEM((8 * C,), jnp.float32),
          pltpu.VMEM((ZR, D), jnp.float32),
          pltpu.VMEM((DENT,), jnp.float32),
          pltpu.VMEM((AVST,), jnp.float32),
          pltpu.VMEM_SHARED((AE,), jnp.float32),
          pltpu.VMEM_SHARED((NPAD, D), jnp.float32),
          pltpu.VMEM_SHARED((DEN,), jnp.float32),
          pltpu.SemaphoreType.DMA,
          pltpu.SemaphoreType.DMA,
      ],
  )
  return kern(src, dst, a_flat, z_all)


def _sc_final_body(sn_hbm, shp_hbm, denp_hbm, out_hbm,
                   snv, snv2, ib, r0, r1, dv, dsum, sem):
  c = lax.axis_index("c")
  s = lax.axis_index("s")
  wid = s * NC + c
  base = wid * SNW

  pltpu.sync_copy(sn_hbm.at[pl.ds(base, SNW)], snv)

  def _shift(i, _):
    snv2[pl.ds(16 * i, 16)] = snv[pl.ds(16 * i, 16)] + NPAD
    return 0
  lax.fori_loop(0, SNW // 16, _shift, 0)

  pltpu.async_copy(shp_hbm.at[snv], r0, sem).wait()
  pltpu.async_copy(shp_hbm.at[snv2], r1, sem).wait()

  for b in range(2 * NUM_HEADS):
    def _mkidx(i, _, b=b):
      ib[pl.ds(16 * i, 16)] = snv[pl.ds(16 * i, 16)] + b * NPAD
      return 0
    lax.fori_loop(0, SNW // 16, _mkidx, 0)
    pltpu.async_copy(denp_hbm.at[ib], dv.at[pl.ds(b * SNW, SNW)], sem).wait()

  def _dcomb(i, _):
    dsum[pl.ds(16 * i, 16)] = (dv[pl.ds(16 * i, 16)] +
                               dv[pl.ds(NUM_HEADS * SNW + 16 * i, 16)])
    return 0
  lax.fori_loop(0, NUM_HEADS * SNW // 16, _dcomb, 0)

  def _row(e, _):
    for hh in range(NUM_HEADS):
      db = plsc.load_gather(dsum, [_full16(hh * SNW) + e])
      db = jnp.where(db == 0.0, 1.0, db)
      for k in range(2):
        sl = pl.ds(32 * hh + 16 * k, 16)
        r0[e, sl] = (r0[e, sl] + r1[e, sl]) / db
    return 0
  lax.fori_loop(0, SNW, _row, 0)

  pltpu.sync_copy(r0, out_hbm.at[pl.ds(base, SNW)])


def _sc_final(sn, shp, denp):
  kern = pl.kernel(
      _sc_final_body,
      out_type=jax.ShapeDtypeStruct((SN_PAD, D), jnp.float32),
      mesh=_mesh(),
      compiler_params=_SC_PARAMS,
      scratch_types=[
          pltpu.VMEM((SNW,), jnp.int32),
          pltpu.VMEM((SNW,), jnp.int32),
          pltpu.VMEM((SNW,), jnp.int32),
          pltpu.VMEM((SNW, D), jnp.float32),
          pltpu.VMEM((SNW, D), jnp.float32),
          pltpu.VMEM((2 * NUM_HEADS * SNW,), jnp.float32),
          pltpu.VMEM((NUM_HEADS * SNW,), jnp.float32),
          pltpu.SemaphoreType.DMA,
      ],
  )
  return kern(sn, shp, denp)


def kernel(h, edge_index, s_node, W, A):
  src = edge_index[0].astype(jnp.int32)
  dst = edge_index[1].astype(jnp.int32)

  wt = W.reshape(D, IN_DIM).T
  ab = jnp.zeros((D, 8), jnp.float32)
  for hh in range(NUM_HEADS):
    ab = ab.at[32 * hh:32 * hh + 32, hh].set(A[hh, :OUT_DIM])
    ab = ab.at[32 * hh:32 * hh + 32, 4 + hh].set(A[hh, OUT_DIM:])

  z_all, a_hm = _tc_prep(h, wt, ab)
  shp, denp = _sc_edge(src, dst, a_hm.reshape(-1), z_all)

  sn = jnp.pad(s_node.astype(jnp.int32), (0, SN_PAD - N_SNODE))
  outp = _sc_final(sn, shp, denp)
  return outp[:N_SNODE]


# speedup vs baseline: 35.5974x; 1.0441x over previous
import functools

import jax
import jax.numpy as jnp
from jax import lax
from jax.experimental import pallas as pl
from jax.experimental.pallas import tpu as pltpu
from jax.experimental.pallas import tpu_sc as plsc

N_NODES = 10000
N_EDGES = 320000
IN_DIM = 128
OUT_DIM = 32
NUM_HEADS = 4
N_SNODE = 5000
D = NUM_HEADS * OUT_DIM

NC = 2
NS = 16
NW = NC * NS
EW = N_EDGES // NW
C = 80
NCHUNK = EW // C
NPAD = 10240
RPT = NPAD // NS
ZR = 64
DEN = NUM_HEADS * NPAD
DENT = DEN // NS
AE = 8 * N_NODES
AET = AE // NS
AVST = 1000

SN_PAD = 5120
SNW = SN_PAD // NW

_mesh = functools.partial(
    plsc.VectorSubcoreMesh, core_axis_name="c", subcore_axis_name="s",
    num_cores=NC, num_subcores=NS)

_SC_PARAMS = pltpu.CompilerParams(needs_layout_passes=False)

_Z16 = functools.partial(jnp.zeros, (16,), jnp.float32)


def _tc_prep_body(h_ref, wt_ref, ab_ref, z_ref, a_ref):
  z = jnp.dot(h_ref[...], wt_ref[...], preferred_element_type=jnp.float32)
  z_ref[...] = z
  a_ref[...] = jnp.dot(z, ab_ref[...], preferred_element_type=jnp.float32).T


def _tc_prep(h, wt, ab):
  return pl.pallas_call(
      _tc_prep_body,
      out_shape=(
          jax.ShapeDtypeStruct((N_NODES, D), jnp.float32),
          jax.ShapeDtypeStruct((8, N_NODES), jnp.float32),
      ),
  )(h, wt, ab)


def _full16(v):
  return jnp.full((16,), v, jnp.int32)


def _sc_edge_body(src_hbm, dst_hbm, a_hbm, z_hbm, shp_hbm, denp_hbm,
                  srcv, dstv, ib0, ib1, ib2, ib3, ib4, ib5, ib6, ib7,
                  rows_v, p_v, av_g, zb1, zb2, avst,
                  a_sp, sh_sp, den_sp, semz, sema):
  ibs = [ib0, ib1, ib2, ib3, ib4, ib5, ib6, ib7]
  idxb = ib0
  c = lax.axis_index("c")
  s = lax.axis_index("s")
  wid = s * NC + c

  def _zero_body(i, _):
    for j in range(8):
      zb1[i, pl.ds(16 * j, 16)] = _Z16()
    return 0
  lax.fori_loop(0, ZR, _zero_body, 0)

  def _zero_flat(i, _):
    zb2[pl.ds(16 * i, 16)] = _Z16()
    return 0
  lax.fori_loop(0, DENT // 16, _zero_flat, 0)

  for k in range(RPT // ZR):
    pltpu.sync_copy(zb1, sh_sp.at[pl.ds(RPT * s + ZR * k, ZR)])
  pltpu.sync_copy(zb2, den_sp.at[pl.ds(DENT * s, DENT)])

  for k in range(AET // AVST):
    a0 = AET * s + AVST * k
    pltpu.sync_copy(a_hbm.at[pl.ds(a0, AVST)], avst)
    pltpu.sync_copy(avst, a_sp.at[pl.ds(a0, AVST)])
  plsc.subcore_barrier()

  base = wid * EW

  def _chunk_body(i, _):
    off = base + i * C
    pltpu.sync_copy(src_hbm.at[pl.ds(off, C)], srcv)
    pltpu.sync_copy(dst_hbm.at[pl.ds(off, C)], dstv)
    cz = pltpu.async_copy(z_hbm.at[srcv], rows_v, semz)

    for b in range(8):
      basev = srcv if b < 4 else dstv
      def _mkidx(g, _, b=b, basev=basev):
        ibs[b][pl.ds(16 * g, 16)] = basev[pl.ds(16 * g, 16)] + b * N_NODES
        return 0
      lax.fori_loop(0, C // 16, _mkidx, 0)
    ca = [pltpu.async_copy(a_sp.at[ibs[b]], av_g.at[pl.ds(C * b, C)], sema)
          for b in range(8)]
    for h_ in ca:
      h_.wait()

    for g in range(C // 16):
      for hh in range(NUM_HEADS):
        u = (av_g[pl.ds(C * hh + 16 * g, 16)] +
             av_g[pl.ds(C * (4 + hh) + 16 * g, 16)])
        p_v[pl.ds(C * hh + 16 * g, 16)] = jnp.exp(
            jnp.where(u >= 0, u, 0.01 * u))

    cz.wait()

    def _group_body(g, _):
      e0 = 16 * g
      for j in range(16):
        for hh in range(NUM_HEADS):
          pb = plsc.load_gather(p_v, [_full16(C * hh + j) + e0])
          for k in range(2):
            sl = pl.ds(32 * hh + 16 * k, 16)
            rows_v[e0 + j, sl] = rows_v[e0 + j, sl] * pb
      return 0
    lax.fori_loop(0, C // 16, _group_body, 0)

    pltpu.sync_copy(rows_v, sh_sp.at[dstv], add=True)
    for hh in range(NUM_HEADS):
      def _mkidx2(g, _, hh=hh):
        idxb[pl.ds(16 * g, 16)] = dstv[pl.ds(16 * g, 16)] + hh * NPAD
        return 0
      lax.fori_loop(0, C // 16, _mkidx2, 0)
      pltpu.sync_copy(p_v.at[pl.ds(C * hh, C)], den_sp.at[idxb], add=True)
    return 0
  lax.fori_loop(0, NCHUNK, _chunk_body, 0)

  plsc.subcore_barrier()

  for k in range(RPT // ZR):
    r0 = RPT * s + ZR * k
    pltpu.sync_copy(sh_sp.at[pl.ds(r0, ZR)], zb1)
    pltpu.sync_copy(zb1, shp_hbm.at[pl.ds(c * NPAD + r0, ZR)])
  pltpu.sync_copy(den_sp.at[pl.ds(DENT * s, DENT)], zb2)
  pltpu.sync_copy(zb2, denp_hbm.at[pl.ds(c * DEN + DENT * s, DENT)])


def _sc_edge(src, dst, a_flat, z_all):
  kern = pl.kernel(
      _sc_edge_body,
      out_type=(
          jax.ShapeDtypeStruct((NC * NPAD, D), jnp.float32),
          jax.ShapeDtypeStruct((NC * DEN,), jnp.float32),
      ),
      mesh=_mesh(),
      compiler_params=_SC_PARAMS,
      scratch_types=[
          pltpu.VMEM((C,), jnp.int32),
          pltpu.VMEM((C,), jnp.int32),
          pltpu.VMEM((C,), jnp.int32),
          pltpu.VMEM((C,), jnp.int32),
          pltpu.VMEM((C,), jnp.int32),
          pltpu.VMEM((C,), jnp.int32),
          pltpu.VMEM((C,), jnp.int32),
          pltpu.VMEM((C,), jnp.int32),
          pltpu.VMEM((C,), jnp.int32),
          pltpu.VMEM((C,), jnp.int32),
          pltpu.VMEM((C, D), jnp.float32),
          pltpu.VMEM((NUM_HEADS * C,), jnp.float32),
          pltpu.VMEM((8 * C,), jnp.float32),
          pltpu.VMEM((ZR, D), jnp.float32),
          pltpu.VMEM((DENT,), jnp.float32),
          pltpu.VMEM((AVST,), jnp.float32),
          pltpu.VMEM_SHARED((AE,), jnp.float32),
          pltpu.VMEM_SHARED((NPAD, D), jnp.float32),
          pltpu.VMEM_SHARED((DEN,), jnp.float32),
          pltpu.SemaphoreType.DMA,
          pltpu.SemaphoreType.DMA,
      ],
  )
  return kern(src, dst, a_flat, z_all)


def _sc_final_body(sn_hbm, shp_hbm, denp_hbm, out_hbm,
                   snv, snv2, ib, r0, r1, dv, dsum, sem):
  c = lax.axis_index("c")
  s = lax.axis_index("s")
  wid = s * NC + c
  base = wid * SNW

  pltpu.sync_copy(sn_hbm.at[pl.ds(base, SNW)], snv)

  def _shift(i, _):
    snv2[pl.ds(16 * i, 16)] = snv[pl.ds(16 * i, 16)] + NPAD
    return 0
  lax.fori_loop(0, SNW // 16, _shift, 0)

  pltpu.async_copy(shp_hbm.at[snv], r0, sem).wait()
  pltpu.async_copy(shp_hbm.at[snv2], r1, sem).wait()

  for b in range(2 * NUM_HEADS):
    def _mkidx(i, _, b=b):
      ib[pl.ds(16 * i, 16)] = snv[pl.ds(16 * i, 16)] + b * NPAD
      return 0
    lax.fori_loop(0, SNW // 16, _mkidx, 0)
    pltpu.async_copy(denp_hbm.at[ib], dv.at[pl.ds(b * SNW, SNW)], sem).wait()

  def _dcomb(i, _):
    dsum[pl.ds(16 * i, 16)] = (dv[pl.ds(16 * i, 16)] +
                               dv[pl.ds(NUM_HEADS * SNW + 16 * i, 16)])
    return 0
  lax.fori_loop(0, NUM_HEADS * SNW // 16, _dcomb, 0)

  def _row(e, _):
    for hh in range(NUM_HEADS):
      db = plsc.load_gather(dsum, [_full16(hh * SNW) + e])
      db = jnp.where(db == 0.0, 1.0, db)
      for k in range(2):
        sl = pl.ds(32 * hh + 16 * k, 16)
        r0[e, sl] = (r0[e, sl] + r1[e, sl]) / db
    return 0
  lax.fori_loop(0, SNW, _row, 0)

  pltpu.sync_copy(r0, out_hbm.at[pl.ds(base, SNW)])


def _sc_final(sn, shp, denp):
  kern = pl.kernel(
      _sc_final_body,
      out_type=jax.ShapeDtypeStruct((SN_PAD, D), jnp.float32),
      mesh=_mesh(),
      compiler_params=_SC_PARAMS,
      scratch_types=[
          pltpu.VMEM((SNW,), jnp.int32),
          pltpu.VMEM((SNW,), jnp.int32),
          pltpu.VMEM((SNW,), jnp.int32),
          pltpu.VMEM((SNW, D), jnp.float32),
          pltpu.VMEM((SNW, D), jnp.float32),
          pltpu.VMEM((2 * NUM_HEADS * SNW,), jnp.float32),
          pltpu.VMEM((NUM_HEADS * SNW,), jnp.float32),
          pltpu.SemaphoreType.DMA,
      ],
  )
  return kern(sn, shp, denp)


def kernel(h, edge_index, s_node, W, A):
  src = edge_index[0].astype(jnp.int32)
  dst = edge_index[1].astype(jnp.int32)

  wt = W.reshape(D, IN_DIM).T
  ab = jnp.zeros((D, 8), jnp.float32)
  for hh in range(NUM_HEADS):
    ab = ab.at[32 * hh:32 * hh + 32, hh].set(A[hh, :OUT_DIM])
    ab = ab.at[32 * hh:32 * hh + 32, 4 + hh].set(A[hh, OUT_DIM:])

  z_all, a_hm = _tc_prep(h, wt, ab)
  shp, denp = _sc_edge(src, dst, a_hm.reshape(-1), z_all)

  sn = jnp.pad(s_node.astype(jnp.int32), (0, SN_PAD - N_SNODE))
  outp = _sc_final(sn, shp, denp)
  return outp[:N_SNODE]
